# baseline (speedup 1.0000x reference)
import functools

import jax
import jax.numpy as jnp
from jax import lax
from jax.experimental import pallas as pl
from jax.experimental.pallas import tpu as pltpu
from jax.experimental.pallas import tpu_sc as plsc

C = 3
T = 64
TS = T // 4
H = 224
W = 224
CH = H // 2
NW = 32
CHUNKS = C * T * 2
PER_W = CHUNKS // NW
NB = 4


def _fast_copy_sc(frames):
    mesh = plsc.VectorSubcoreMesh(core_axis_name="c", subcore_axis_name="s")

    @functools.partial(
        pl.kernel,
        out_type=jax.ShapeDtypeStruct((C, T, H, W), jnp.float32),
        mesh=mesh,
        scratch_types=[
            [pltpu.VMEM((CH, W), jnp.float32) for _ in range(NB)],
            [pltpu.SemaphoreType.DMA for _ in range(NB)],
            [pltpu.SemaphoreType.DMA for _ in range(NB)],
        ],
        compiler_params=pltpu.CompilerParams(use_tc_tiling_on_sc=True),
    )
    def k(src, fast, bufs, in_sems, out_sems):
        wid = lax.axis_index("s") * 2 + lax.axis_index("c")
        base = wid * PER_W

        def coords(i):
            m = base + i
            u = m // 2
            half = m % 2
            return u // T, u % T, half

        def src_sl(c, t, half):
            return src.at[c, t, pl.ds(half * CH, CH), :]

        def fast_sl(c, t, half):
            return fast.at[c, t, pl.ds(half * CH, CH), :]

        for p in range(NB - 1):
            c, t, half = coords(p)
            pltpu.async_copy(src_sl(c, t, half), bufs[p], in_sems[p])

        for i in range(PER_W):
            b = i % NB
            c, t, half = coords(i)
            pltpu.make_async_copy(src_sl(c, t, half), bufs[b], in_sems[b]).wait()
            pltpu.async_copy(bufs[b], fast_sl(c, t, half), out_sems[b])

            nxt = i + NB - 1
            if nxt < PER_W:
                if i >= 1:
                    pc, pt, ph = coords(i - 1)
                    pb = (i - 1) % NB
                    pltpu.make_async_copy(
                        bufs[pb], fast_sl(pc, pt, ph), out_sems[pb]).wait()
                cn, tn, hn = coords(nxt)
                pltpu.async_copy(src_sl(cn, tn, hn), bufs[nxt % NB], in_sems[nxt % NB])

        for i in range(PER_W - NB, PER_W):
            c, t, half = coords(i)
            b = i % NB
            pltpu.make_async_copy(bufs[b], fast_sl(c, t, half), out_sems[b]).wait()

    return k(frames)


def _slow_gather_tc(frames):
    def body(src_ref, out_ref):
        out_ref[...] = src_ref[...]

    return pl.pallas_call(
        body,
        grid=(C * TS,),
        in_specs=[
            pl.BlockSpec(
                (1, 1, H, W),
                lambda i: (i // TS, (21 * (i % TS)) // 5, 0, 0),
            )
        ],
        out_specs=pl.BlockSpec(
            (1, 1, H, W),
            lambda i: (i // TS, i % TS, 0, 0),
        ),
        out_shape=jax.ShapeDtypeStruct((C, TS, H, W), jnp.float32),
    )(frames)


def kernel(frames):
    fast = _fast_copy_sc(frames)
    slow = _slow_gather_tc(frames)
    return (slow, fast)


# speedup vs baseline: 1.1286x; 1.1286x over previous
import functools

import jax
import jax.numpy as jnp
from jax import lax
from jax.experimental import pallas as pl
from jax.experimental.pallas import tpu as pltpu
from jax.experimental.pallas import tpu_sc as plsc

C = 3
T = 64
TS = T // 4
H = 224
W = 224
QH = H // 4
NW = 32
PER_W = 24
NB = 8


def _pack(frames):
    mesh = plsc.VectorSubcoreMesh(core_axis_name="c", subcore_axis_name="s")

    @functools.partial(
        pl.kernel,
        out_type=(
            jax.ShapeDtypeStruct((C, TS, H, W), jnp.float32),
            jax.ShapeDtypeStruct((C, T, H, W), jnp.float32),
        ),
        mesh=mesh,
        scratch_types=[
            [pltpu.VMEM((QH, W), jnp.float32) for _ in range(NB)],
            [pltpu.SemaphoreType.DMA for _ in range(NB)],
            [pltpu.SemaphoreType.DMA for _ in range(NB)],
        ],
        compiler_params=pltpu.CompilerParams(use_tc_tiling_on_sc=True),
    )
    def k(src, slow, fast, bufs, in_sems, out_sems):
        wid = lax.axis_index("s") * 2 + lax.axis_index("c")

        def coords(i):
            if i % 4 == 0:
                s = wid * 6 + i // 4
                c = s // 64
                j = (s // 4) % TS
                t = (21 * j) // 5
                qtr = s % 4
                return c, t, qtr, j
            d = wid * 18 + (i - 1 - i // 4)
            c = d // 192
            r = (d // 4) % 48
            q = r % 16
            t = 21 * (r // 16) + q + 1 + q // 3 - q // 15
            qtr = d % 4
            return c, t, qtr, None

        def src_sl(c, t, qtr):
            return src.at[c, t, pl.ds(qtr * QH, QH), :]

        def fast_sl(c, t, qtr):
            return fast.at[c, t, pl.ds(qtr * QH, QH), :]

        def slow_sl(c, j, qtr):
            return slow.at[c, j, pl.ds(qtr * QH, QH), :]

        def issue_out(i):
            c, t, qtr, j = coords(i)
            b = i % NB
            pltpu.async_copy(bufs[b], fast_sl(c, t, qtr), out_sems[b])
            if j is not None:
                pltpu.async_copy(bufs[b], slow_sl(c, j, qtr), out_sems[b])

        def wait_out(i):
            c, t, qtr, j = coords(i)
            b = i % NB
            pltpu.make_async_copy(bufs[b], fast_sl(c, t, qtr), out_sems[b]).wait()
            if j is not None:
                pltpu.make_async_copy(bufs[b], slow_sl(c, j, qtr), out_sems[b]).wait()

        for p in range(NB - 1):
            c, t, qtr, _ = coords(p)
            pltpu.async_copy(src_sl(c, t, qtr), bufs[p], in_sems[p])

        for i in range(PER_W):
            b = i % NB
            c, t, qtr, _ = coords(i)
            pltpu.make_async_copy(src_sl(c, t, qtr), bufs[b], in_sems[b]).wait()
            issue_out(i)

            nxt = i + NB - 1
            if nxt < PER_W:
                if i >= 1:
                    wait_out(i - 1)
                cn, tn, qn, _ = coords(nxt)
                pltpu.async_copy(src_sl(cn, tn, qn), bufs[nxt % NB], in_sems[nxt % NB])

        for i in range(PER_W - NB, PER_W):
            wait_out(i)

    return k(frames)


def kernel(frames):
    slow, fast = _pack(frames)
    return (slow, fast)


# speedup vs baseline: 1.1580x; 1.0261x over previous
import functools

import jax
import jax.numpy as jnp
from jax import lax
from jax.experimental import pallas as pl
from jax.experimental.pallas import tpu as pltpu
from jax.experimental.pallas import tpu_sc as plsc

C = 3
T = 64
TS = T // 4
H = 224
W = 224
CH = H // 2
NW = 32
PER_W = (C * TS * 2) // NW
NB = 3
FB = 4


def _slow_gather_sc(frames):
    mesh = plsc.VectorSubcoreMesh(core_axis_name="c", subcore_axis_name="s")

    @functools.partial(
        pl.kernel,
        out_type=jax.ShapeDtypeStruct((C, TS, H, W), jnp.float32),
        mesh=mesh,
        scratch_types=[
            [pltpu.VMEM((CH, W), jnp.float32) for _ in range(NB)],
            [pltpu.SemaphoreType.DMA for _ in range(NB)],
            [pltpu.SemaphoreType.DMA for _ in range(NB)],
        ],
        compiler_params=pltpu.CompilerParams(use_tc_tiling_on_sc=True),
    )
    def k(src, slow, bufs, in_sems, out_sems):
        wid = lax.axis_index("s") * 2 + lax.axis_index("c")

        def coords(i):
            m = wid * PER_W + i
            u = m // 2
            half = m % 2
            c = u // TS
            j = u % TS
            t = (21 * j) // 5
            return c, t, half, j

        def src_sl(c, t, half):
            return src.at[c, t, pl.ds(half * CH, CH), :]

        def slow_sl(c, j, half):
            return slow.at[c, j, pl.ds(half * CH, CH), :]

        for p in range(NB - 1):
            c, t, half, _ = coords(p)
            pltpu.async_copy(src_sl(c, t, half), bufs[p], in_sems[p])

        for i in range(PER_W):
            b = i % NB
            c, t, half, j = coords(i)
            pltpu.make_async_copy(src_sl(c, t, half), bufs[b], in_sems[b]).wait()
            pltpu.async_copy(bufs[b], slow_sl(c, j, half), out_sems[b])
            nxt = i + NB - 1
            if nxt < PER_W:
                if i >= 1:
                    pc, pt, ph, pj = coords(i - 1)
                    pb = (i - 1) % NB
                    pltpu.make_async_copy(
                        bufs[pb], slow_sl(pc, pj, ph), out_sems[pb]).wait()
                cn, tn, hn, _ = coords(nxt)
                pltpu.async_copy(src_sl(cn, tn, hn), bufs[nxt % NB], in_sems[nxt % NB])

        for i in range(max(0, PER_W - NB), PER_W):
            c, t, half, j = coords(i)
            b = i % NB
            pltpu.make_async_copy(bufs[b], slow_sl(c, j, half), out_sems[b]).wait()

    return k(frames)


def _fast_copy_tc(frames):
    def body(src_ref, out_ref):
        out_ref[...] = src_ref[...]

    return pl.pallas_call(
        body,
        grid=(T // FB,),
        in_specs=[pl.BlockSpec((C, FB, H, W), lambda i: (0, i, 0, 0))],
        out_specs=pl.BlockSpec((C, FB, H, W), lambda i: (0, i, 0, 0)),
        out_shape=jax.ShapeDtypeStruct((C, T, H, W), jnp.float32),
    )(frames)


def kernel(frames):
    slow = _slow_gather_sc(frames)
    fast = _fast_copy_tc(frames)
    return (slow, fast)
